# Initial kernel scaffold; baseline (speedup 1.0000x reference)
import jax
import jax.numpy as jnp
from jax.experimental import pallas as pl


def kernel(embeddings, edge_index, adj_values, W1, W2):
    raise NotImplementedError("write your pallas kernel here")


# speedup vs baseline: 4.7767x; 4.7767x over previous
import functools

import jax
import jax.numpy as jnp
from jax import lax
from jax.experimental import pallas as pl
from jax.experimental.pallas import tpu as pltpu
from jax.experimental.pallas import tpu_sc as plsc


def _sc_segment_sum(emb, src2d, dst2d, adj2d, zeros):
    n, d = emb.shape
    nrows, b = src2d.shape
    info = plsc.get_sparse_core_info()
    nc, ns, lanes = info.num_cores, info.num_subcores, info.num_lanes
    nw = nc * ns
    rows_per_sub = (n // ns) // 8 * 8
    tail_start = ns * rows_per_sub
    tail = n - tail_start

    mesh = plsc.VectorSubcoreMesh(core_axis_name="c", subcore_axis_name="s")

    @functools.partial(
        pl.kernel,
        mesh=mesh,
        out_type=jax.ShapeDtypeStruct((nc, n, d), jnp.float32),
        compiler_params=pltpu.CompilerParams(needs_layout_passes=False),
        scratch_types=[
            pltpu.VMEM((b,), jnp.int32),
            pltpu.VMEM((b,), jnp.int32),
            pltpu.VMEM((b,), jnp.float32),
            pltpu.VMEM((b, d), jnp.float32),
            pltpu.VMEM_SHARED((n, d), jnp.float32),
            pltpu.SemaphoreType.DMA,
        ],
    )
    def sc_k(emb_hbm, src_hbm, dst_hbm, adj_hbm, zeros_hbm, part_hbm,
             src_v, dst_v, adj_v, rows_v, acc, sem):
        c = lax.axis_index("c")
        s = lax.axis_index("s")
        w = c * ns + s

        off = pl.multiple_of(s * rows_per_sub, 8)
        pltpu.sync_copy(
            zeros_hbm.at[pl.ds(off, rows_per_sub)],
            acc.at[pl.ds(off, rows_per_sub)],
        )
        if tail:
            @pl.when(s == 0)
            def _zero_tail():
                pltpu.sync_copy(
                    zeros_hbm.at[pl.ds(tail_start, tail)],
                    acc.at[pl.ds(tail_start, tail)],
                )
        plsc.subcore_barrier()

        base = nrows // nw
        rem = nrows % nw
        cnt = base + (w < rem).astype(jnp.int32)
        start = w * base + jnp.minimum(w, rem)

        def chunk_body(i, carry):
            row = start + i
            pltpu.sync_copy(src_hbm.at[row], src_v)
            pltpu.sync_copy(dst_hbm.at[row], dst_v)
            pltpu.sync_copy(adj_hbm.at[row], adj_v)
            pltpu.async_copy(emb_hbm.at[src_v], rows_v, sem).wait()

            def scale_body(r, carry2):
                av = plsc.load_gather(
                    adj_v, [jnp.full((lanes,), r, jnp.int32)]
                )
                for jj in range(d // lanes):
                    sl = pl.ds(jj * lanes, lanes)
                    rows_v[r, sl] = rows_v[r, sl] * av
                return carry2

            lax.fori_loop(0, b, scale_body, 0)
            pltpu.sync_copy(rows_v, acc.at[dst_v], add=True)
            return carry

        lax.fori_loop(0, cnt, chunk_body, 0)
        plsc.subcore_barrier()

        pltpu.sync_copy(
            acc.at[pl.ds(off, rows_per_sub)],
            part_hbm.at[c, pl.ds(off, rows_per_sub)],
        )
        if tail:
            @pl.when(s == 0)
            def _write_tail():
                pltpu.sync_copy(
                    acc.at[pl.ds(tail_start, tail)],
                    part_hbm.at[c, pl.ds(tail_start, tail)],
                )

    return sc_k(emb, src2d, dst2d, adj2d, zeros)


def _tc_transform(partials, emb, w1, w2):
    n, d = emb.shape
    nc = partials.shape[0]
    blk = 2000

    def body(p_ref, e_ref, w1_ref, w2_ref, o_ref):
        rel = p_ref[0]
        for i in range(1, nc):
            rel = rel + p_ref[i]
        o_ref[...] = jnp.dot(
            rel, w1_ref[...], preferred_element_type=jnp.float32
        ) + jnp.dot(
            rel * e_ref[...], w2_ref[...], preferred_element_type=jnp.float32
        )

    return pl.pallas_call(
        body,
        grid=(n // blk,),
        in_specs=[
            pl.BlockSpec((nc, blk, d), lambda i: (0, i, 0)),
            pl.BlockSpec((blk, d), lambda i: (i, 0)),
            pl.BlockSpec((d, d), lambda i: (0, 0)),
            pl.BlockSpec((d, d), lambda i: (0, 0)),
        ],
        out_specs=pl.BlockSpec((blk, d), lambda i: (i, 0)),
        out_shape=jax.ShapeDtypeStruct((n, d), jnp.float32),
    )(partials, emb, w1, w2)


def kernel(embeddings, edge_index, adj_values, W1, W2):
    n, d = embeddings.shape
    e = adj_values.shape[0]
    b = 128
    dst2d = edge_index[0].reshape(e // b, b)
    src2d = edge_index[1].reshape(e // b, b)
    adj2d = adj_values.reshape(e // b, b)
    zeros = jnp.zeros_like(embeddings)
    partials = _sc_segment_sum(embeddings, src2d, dst2d, adj2d, zeros)
    return _tc_transform(partials, embeddings, W1, W2)
